# baseline (speedup 1.0000x reference)
import functools

import jax
import jax.numpy as jnp
import numpy as np
from jax import lax
from jax.experimental import pallas as pl
from jax.experimental.pallas import tpu as pltpu
from jax.experimental.pallas import tpu_sc as plsc

VOCAB = 1000000
D = 64
DP = 128
BATCH = 4096
SEQ = 200
B_FLAT = BATCH * SEQ

NUM_WORKERS = 32
ROWS_PER_W = B_FLAT // NUM_WORKERS
CHUNK = 320
N_CHUNKS = ROWS_PER_W // CHUNK
GSLICE = 80
N_GS = CHUNK // GSLICE
LANES = 16
NBUF = 4


def _sinusoidal_pe_np(max_len, d_model):
    pos = np.arange(max_len, dtype=np.float32)[:, None]
    div = np.exp(np.arange(0, d_model, 2, dtype=np.float32) * (-np.log(10000.0) / d_model))
    pe = np.zeros((max_len, d_model), dtype=np.float32)
    pe[:, 0::2] = np.sin(pos * div)
    pe[:, 1::2] = np.cos(pos * div)
    return pe


_PE = _sinusoidal_pe_np(SEQ, D)


def _emb_body(table_hbm, idx_hbm, pe_hbm, out_hbm, idx_v, rows_v, pe_v,
              g0, g1, g2, g3, w0, w1, w2, w3):
    wid = lax.axis_index("s") * 2 + lax.axis_index("c")
    base = wid * ROWS_PER_W
    gsems = (g0, g1, g2, g3)
    wsems = (w0, w1, w2, w3)

    pltpu.sync_copy(pe_hbm, pe_v)

    def row_of(c):
        return base + c * CHUNK

    def fire(buf, row0):
        pltpu.sync_copy(idx_hbm.at[pl.ds(row0, CHUNK)], idx_v.at[buf])
        for s in range(N_GS):
            pltpu.async_copy(
                table_hbm.at[idx_v.at[buf].at[pl.ds(s * GSLICE, GSLICE)]],
                rows_v.at[buf].at[pl.ds(s * GSLICE, GSLICE)],
                gsems[buf],
            )

    def drain(buf):
        for s in range(N_GS):
            pltpu.make_async_copy(
                table_hbm.at[idx_v.at[buf].at[pl.ds(s * GSLICE, GSLICE)]],
                rows_v.at[buf].at[pl.ds(s * GSLICE, GSLICE)],
                gsems[buf],
            ).wait()

    def pe_add(buf, row0):
        m0 = lax.rem(row0, SEQ)

        def pe_row(r, _):
            m = lax.rem(m0 + r, SEQ)
            for col in range(D // LANES):
                plsc.addupdate(
                    rows_v.at[buf, r, pl.ds(col * LANES, LANES)],
                    pe_v[m, pl.ds(col * LANES, LANES)],
                )
            return 0

        lax.fori_loop(0, CHUNK, pe_row, 0)

    def wfire(buf, row0):
        pltpu.async_copy(
            rows_v.at[buf],
            out_hbm.at[pl.ds(row0, CHUNK)].at[:, pl.ds(0, D)],
            wsems[buf],
        )

    def wwait(buf, row0):
        pltpu.make_async_copy(
            rows_v.at[buf],
            out_hbm.at[pl.ds(row0, CHUNK)].at[:, pl.ds(0, D)],
            wsems[buf],
        ).wait()

    def step(c, waitw, firenext):
        b = c % NBUF
        drain(b)
        pe_add(b, row_of(c))
        wfire(b, row_of(c))
        if firenext:
            bn = (c + 2) % NBUF
            if waitw:
                wwait(bn, row_of(c - 2))
            fire(bn, row_of(c + 2))

    fire(0, row_of(0))
    fire(1, row_of(1))
    step(0, False, True)
    step(1, False, True)
    step(2, True, True)
    step(3, True, True)

    def group(g, _):
        c0 = 4 * g

        def gstep(k):
            c = c0 + k
            b = k
            drain(b)
            pe_add(b, row_of(c))
            wfire(b, row_of(c))
            bn = (k + 2) % NBUF
            wwait(bn, row_of(c - 2))
            fire(bn, row_of(c + 2))

        gstep(0)
        gstep(1)
        gstep(2)
        gstep(3)
        return 0

    lax.fori_loop(1, 19, group, 0)

    step(76, True, True)
    step(77, True, True)
    step(78, False, False)
    step(79, False, False)
    for c in (76, 77, 78, 79):
        wwait(c % NBUF, row_of(c))


_mesh = plsc.VectorSubcoreMesh(core_axis_name="c", subcore_axis_name="s")

_emb = functools.partial(
    pl.kernel,
    mesh=_mesh,
    out_type=jax.ShapeDtypeStruct((B_FLAT, DP), jnp.float32),
    compiler_params=pltpu.CompilerParams(use_tc_tiling_on_sc=False),
    scratch_types=[
        pltpu.VMEM((NBUF, CHUNK), jnp.int32),
        pltpu.VMEM((NBUF, CHUNK, D), jnp.float32),
        pltpu.VMEM((SEQ, D), jnp.float32),
        pltpu.SemaphoreType.DMA,
        pltpu.SemaphoreType.DMA,
        pltpu.SemaphoreType.DMA,
        pltpu.SemaphoreType.DMA,
        pltpu.SemaphoreType.DMA,
        pltpu.SemaphoreType.DMA,
        pltpu.SemaphoreType.DMA,
        pltpu.SemaphoreType.DMA,
    ],
)(_emb_body)


def kernel(input, table):
    idx = input.reshape(B_FLAT).astype(jnp.int32)
    out = _emb(table, idx, jnp.asarray(_PE))
    return out[:, :D].reshape(BATCH, SEQ, D)


# speedup vs baseline: 1.2237x; 1.2237x over previous
import functools

import jax
import jax.numpy as jnp
import numpy as np
from jax import lax
from jax.experimental import pallas as pl
from jax.experimental.pallas import tpu as pltpu
from jax.experimental.pallas import tpu_sc as plsc

VOCAB = 1000000
D = 64
DP = 128
BATCH = 4096
SEQ = 200
B_FLAT = BATCH * SEQ

NUM_WORKERS = 32
ROWS_PER_W = B_FLAT // NUM_WORKERS
CHUNK = 400
N_CHUNKS = ROWS_PER_W // CHUNK
GSLICE = 80
N_GS = CHUNK // GSLICE
LANES = 16


def _sinusoidal_pe_np(max_len, d_model):
    pos = np.arange(max_len, dtype=np.float32)[:, None]
    div = np.exp(np.arange(0, d_model, 2, dtype=np.float32) * (-np.log(10000.0) / d_model))
    pe = np.zeros((max_len, d_model), dtype=np.float32)
    pe[:, 0::2] = np.sin(pos * div)
    pe[:, 1::2] = np.cos(pos * div)
    return pe


_PE = _sinusoidal_pe_np(SEQ, D)


def _emb_body(table_hbm, idx_hbm, pe_hbm, out_hbm, idx_v, rows_v, pe_v, sem0, sem1):
    wid = lax.axis_index("s") * 2 + lax.axis_index("c")
    base = wid * ROWS_PER_W
    sems = (sem0, sem1)

    pltpu.sync_copy(pe_hbm, pe_v)

    def fire(buf, row0):
        pltpu.sync_copy(idx_hbm.at[pl.ds(row0, CHUNK)], idx_v.at[buf])
        for s in range(N_GS):
            pltpu.async_copy(
                table_hbm.at[idx_v.at[buf].at[pl.ds(s * GSLICE, GSLICE)]],
                rows_v.at[buf].at[pl.ds(s * GSLICE, GSLICE)],
                sems[buf],
            )

    def drain(buf):
        for s in range(N_GS):
            pltpu.make_async_copy(
                table_hbm.at[idx_v.at[buf].at[pl.ds(s * GSLICE, GSLICE)]],
                rows_v.at[buf].at[pl.ds(s * GSLICE, GSLICE)],
                sems[buf],
            ).wait()

    def finish(buf, row0):
        def pe_row(r, _):
            for col in range(D // LANES):
                pvec = pe_v[r, pl.ds(col * LANES, LANES)]
                for rep in range(CHUNK // SEQ):
                    plsc.addupdate(
                        rows_v.at[buf, rep * SEQ + r, pl.ds(col * LANES, LANES)],
                        pvec,
                    )
            return 0

        lax.fori_loop(0, SEQ, pe_row, 0)
        pltpu.sync_copy(
            rows_v.at[buf],
            out_hbm.at[pl.ds(row0, CHUNK)].at[:, pl.ds(0, D)],
        )

    fire(0, base)

    def pair(i, _):
        a = base + (2 * i) * CHUNK
        fire(1, a + CHUNK)
        drain(0)
        finish(0, a)
        fire(0, a + 2 * CHUNK)
        drain(1)
        finish(1, a + CHUNK)
        return 0

    lax.fori_loop(0, N_CHUNKS // 2 - 1, pair, 0)
    a = base + (N_CHUNKS - 2) * CHUNK
    fire(1, a + CHUNK)
    drain(0)
    finish(0, a)
    drain(1)
    finish(1, a + CHUNK)


_mesh = plsc.VectorSubcoreMesh(core_axis_name="c", subcore_axis_name="s")

_emb = functools.partial(
    pl.kernel,
    mesh=_mesh,
    out_type=jax.ShapeDtypeStruct((B_FLAT, DP), jnp.float32),
    compiler_params=pltpu.CompilerParams(use_tc_tiling_on_sc=False),
    scratch_types=[
        pltpu.VMEM((2, CHUNK), jnp.int32),
        pltpu.VMEM((2, CHUNK, D), jnp.float32),
        pltpu.VMEM((SEQ, D), jnp.float32),
        pltpu.SemaphoreType.DMA,
        pltpu.SemaphoreType.DMA,
    ],
)(_emb_body)


def kernel(input, table):
    idx = input.reshape(B_FLAT).astype(jnp.int32)
    out = _emb(table, idx, jnp.asarray(_PE))
    return out[:, :D].reshape(BATCH, SEQ, D)
